# Initial kernel scaffold; baseline (speedup 1.0000x reference)
import jax
import jax.numpy as jnp
from jax.experimental import pallas as pl


def kernel(atomic_num, hyb, heavydegree, heterodegree, smarts, emb_atomic_num, emb_hyb, emb_heavydegree, emb_heterodegree, emb_smarts, partialcharge, charge_W, charge_b, proj_W, proj_b):
    raise NotImplementedError("write your pallas kernel here")


# speedup vs baseline: 12.8634x; 12.8634x over previous
import functools

import jax
import jax.numpy as jnp
from jax.experimental import pallas as pl

_N = 524288
_BLOCK = 2048
_GAMMA = 10.0

_OFF_C1 = 45
_OFF_C2 = 45 + 30
_OFF_RBF = 75 + 32
_N_RBF = 20


def _body(an_ref, hy_ref, hd_ref, ht_ref, sm_ref, pc_ref, w_ref, b_ref,
          out_ref):
    blk = an_ref.shape[0]
    lane = jax.lax.broadcasted_iota(jnp.int32, (blk, 128), 1)
    lanef = lane.astype(jnp.float32)
    centers = (lanef - float(_OFF_RBF)) * 0.1
    rbf_mask = (lane >= _OFF_RBF) & (lane < _OFF_RBF + _N_RBF)

    def feat(j):
        c0 = an_ref[:, j:j + 1] * 5 + hd_ref[:, j:j + 1]
        c1 = hy_ref[:, j:j + 1] * 5 + ht_ref[:, j:j + 1] + _OFF_C1
        c2 = sm_ref[:, j:j + 1] + _OFF_C2
        hot = (lane == c0) | (lane == c1) | (lane == c2)
        x = pc_ref[:, j:j + 1]
        d = x - centers
        r = jnp.exp(-_GAMMA * d * d)
        return jnp.where(hot, 1.0, jnp.where(rbf_mask, r, 0.0))

    g = jnp.concatenate([feat(0), feat(1)], axis=1)
    out_ref[...] = (
        jnp.dot(g, w_ref[...], preferred_element_type=jnp.float32)
        + b_ref[...])


@functools.partial(jax.jit, static_argnames=("interpret",))
def _run(an, hy, hd, ht, sm, pc, w, bias, *, interpret=False):
    grid = (_N // _BLOCK,)
    ispec = pl.BlockSpec((_BLOCK, 2), lambda i: (i, 0))
    return pl.pallas_call(
        _body,
        grid=grid,
        in_specs=[ispec, ispec, ispec, ispec, ispec, ispec,
                  pl.BlockSpec((256, 128), lambda i: (0, 0)),
                  pl.BlockSpec((1, 128), lambda i: (0, 0))],
        out_specs=pl.BlockSpec((_BLOCK, 128), lambda i: (i, 0)),
        out_shape=jax.ShapeDtypeStruct((_N, 128), jnp.float32),
        interpret=interpret,
    )(an, hy, hd, ht, sm, pc, w, bias)


def kernel(atomic_num, hyb, heavydegree, heterodegree, smarts,
           emb_atomic_num, emb_hyb, emb_heavydegree, emb_heterodegree,
           emb_smarts, partialcharge, charge_W, charge_b, proj_W, proj_b,
           interpret=False):
    P = proj_W.T
    P0, P1 = P[:32], P[32:]
    t01 = (emb_atomic_num[:, None, :]
           + emb_heavydegree[None, :, :]).reshape(45, 32)
    t23 = (emb_hyb[:, None, :]
           + emb_heterodegree[None, :, :]).reshape(30, 32)
    tcat = jnp.concatenate([t01, t23, emb_smarts], axis=0)

    def fused_w(Pj):
        wj = jnp.concatenate([tcat @ Pj, charge_W.T @ Pj], axis=0)
        return jnp.pad(wj, ((0, 1), (0, 0)))

    w = jnp.concatenate([fused_w(P0), fused_w(P1)], axis=0)
    bias = (charge_b @ (P0 + P1) + proj_b).reshape(1, 128)

    return _run(atomic_num, hyb, heavydegree, heterodegree, smarts,
                partialcharge, w, bias, interpret=interpret)


# speedup vs baseline: 12.9275x; 1.0050x over previous
import functools

import jax
import jax.numpy as jnp
from jax.experimental import pallas as pl

_N = 524288
_BLOCK = 2048
_GAMMA = 10.0

_OFF_C1 = 45
_OFF_C2 = 45 + 30
_OFF_RBF = 75 + 32
_N_RBF = 20


def _body(an_ref, hy_ref, hd_ref, ht_ref, sm_ref, pc_ref, w0_ref, w1_ref,
          b_ref, out_ref):
    blk = an_ref.shape[0]
    lane = jax.lax.broadcasted_iota(jnp.int32, (blk, 128), 1)
    lanef = lane.astype(jnp.float32)
    centers = (lanef - float(_OFF_RBF)) * 0.1
    rbf_mask = (lane >= _OFF_RBF) & (lane < _OFF_RBF + _N_RBF)

    def feat(j):
        c0 = an_ref[:, j:j + 1] * 5 + hd_ref[:, j:j + 1]
        c1 = hy_ref[:, j:j + 1] * 5 + ht_ref[:, j:j + 1] + _OFF_C1
        c2 = sm_ref[:, j:j + 1] + _OFF_C2
        hot = (lane == c0) | (lane == c1) | (lane == c2)
        x = pc_ref[:, j:j + 1]
        d = x - centers
        r = jnp.exp(-_GAMMA * d * d)
        return jnp.where(hot, 1.0, jnp.where(rbf_mask, r, 0.0))

    out_ref[...] = (
        jnp.dot(feat(0), w0_ref[...], preferred_element_type=jnp.float32)
        + jnp.dot(feat(1), w1_ref[...], preferred_element_type=jnp.float32)
        + b_ref[...])


@functools.partial(jax.jit, static_argnames=("interpret",))
def _run(an, hy, hd, ht, sm, pc, w0, w1, bias, *, interpret=False):
    grid = (_N // _BLOCK,)
    ispec = pl.BlockSpec((_BLOCK, 2), lambda i: (i, 0))
    wspec = pl.BlockSpec((128, 128), lambda i: (0, 0))
    return pl.pallas_call(
        _body,
        grid=grid,
        in_specs=[ispec, ispec, ispec, ispec, ispec, ispec,
                  wspec, wspec,
                  pl.BlockSpec((1, 128), lambda i: (0, 0))],
        out_specs=pl.BlockSpec((_BLOCK, 128), lambda i: (i, 0)),
        out_shape=jax.ShapeDtypeStruct((_N, 128), jnp.float32),
        interpret=interpret,
    )(an, hy, hd, ht, sm, pc, w0, w1, bias)


def kernel(atomic_num, hyb, heavydegree, heterodegree, smarts,
           emb_atomic_num, emb_hyb, emb_heavydegree, emb_heterodegree,
           emb_smarts, partialcharge, charge_W, charge_b, proj_W, proj_b,
           interpret=False):
    P = proj_W.T
    P0, P1 = P[:32], P[32:]
    t01 = (emb_atomic_num[:, None, :]
           + emb_heavydegree[None, :, :]).reshape(45, 32)
    t23 = (emb_hyb[:, None, :]
           + emb_heterodegree[None, :, :]).reshape(30, 32)
    tcat = jnp.concatenate([t01, t23, emb_smarts], axis=0)

    def fused_w(Pj):
        wj = jnp.concatenate([tcat @ Pj, charge_W.T @ Pj], axis=0)
        return jnp.pad(wj, ((0, 1), (0, 0)))

    bias = (charge_b @ (P0 + P1) + proj_b).reshape(1, 128)

    return _run(atomic_num, hyb, heavydegree, heterodegree, smarts,
                partialcharge, fused_w(P0), fused_w(P1), bias,
                interpret=interpret)


# speedup vs baseline: 23.7469x; 1.8369x over previous
import functools
import math

import jax
import jax.numpy as jnp
from jax.experimental import pallas as pl
import numpy as np

_N = 524288
_BLOCK = 2048
_GAMMA = 10.0
_KAPPA = 8.0
_SQG = math.sqrt(_GAMMA)


def _build_m():
    m = np.zeros((13, 256), dtype=np.float32)
    for j in (0, 1):
        for l in range(45):
            m[0 + j, 128 * j + l] = 5.0 * _KAPPA
            m[4 + j, 128 * j + l] = _KAPPA
            m[12, 128 * j + l] = -_KAPPA * l
        for l in range(45, 75):
            m[2 + j, 128 * j + l] = 5.0 * _KAPPA
            m[6 + j, 128 * j + l] = _KAPPA
            m[12, 128 * j + l] = -_KAPPA * (l - 45)
        for l in range(75, 107):
            m[8 + j, 128 * j + l] = _KAPPA
            m[12, 128 * j + l] = -_KAPPA * (l - 75)
        for l in range(107, 127):
            m[10 + j, 128 * j + l] = _SQG
            m[12, 128 * j + l] = -_SQG * 0.1 * (l - 107)
    return jnp.asarray(m)


_M = _build_m()


def _body(x_ref, m_ref, w_ref, out_ref):
    q = jax.lax.dot_general(
        x_ref[...], m_ref[...], (((1,), (0,)), ((), ())),
        precision=jax.lax.Precision.HIGHEST,
        preferred_element_type=jnp.float32)
    g = jnp.exp(-(q * q))
    out_ref[...] = jnp.dot(g, w_ref[...], preferred_element_type=jnp.float32)


@functools.partial(jax.jit, static_argnames=("interpret",))
def _run(x, m, w, *, interpret=False):
    grid = (_N // _BLOCK,)
    return pl.pallas_call(
        _body,
        grid=grid,
        in_specs=[pl.BlockSpec((_BLOCK, 13), lambda i: (i, 0)),
                  pl.BlockSpec((13, 256), lambda i: (0, 0)),
                  pl.BlockSpec((256, 128), lambda i: (0, 0))],
        out_specs=pl.BlockSpec((_BLOCK, 128), lambda i: (i, 0)),
        out_shape=jax.ShapeDtypeStruct((_N, 128), jnp.float32),
        interpret=interpret,
    )(x, m, w)


def kernel(atomic_num, hyb, heavydegree, heterodegree, smarts,
           emb_atomic_num, emb_hyb, emb_heavydegree, emb_heterodegree,
           emb_smarts, partialcharge, charge_W, charge_b, proj_W, proj_b,
           interpret=False):
    P = proj_W.T
    P0, P1 = P[:32], P[32:]
    t01 = (emb_atomic_num[:, None, :]
           + emb_heavydegree[None, :, :]).reshape(45, 32)
    t23 = (emb_hyb[:, None, :]
           + emb_heterodegree[None, :, :]).reshape(30, 32)
    tcat = jnp.concatenate([t01, t23, emb_smarts], axis=0)

    def fused_w(Pj, bias_row):
        return jnp.concatenate(
            [tcat @ Pj, charge_W.T @ Pj, bias_row.reshape(1, 128)], axis=0)

    w = jnp.concatenate(
        [fused_w(P0, charge_b @ P0 + proj_b), fused_w(P1, charge_b @ P1)],
        axis=0)

    f32 = jnp.float32
    x = jnp.concatenate(
        [atomic_num.astype(f32), hyb.astype(f32), heavydegree.astype(f32),
         heterodegree.astype(f32), smarts.astype(f32), partialcharge,
         jnp.ones((_N, 1), f32)], axis=1)

    return _run(x, _M, w, interpret=interpret)


# speedup vs baseline: 32.8980x; 1.3854x over previous
import functools

import jax
import jax.numpy as jnp
from jax.experimental import pallas as pl
import numpy as np

_N = 524288
_BLOCK = 2048
_GAMMA = 10.0
_KAPPA = 8.0
_A = 3.15625
_S = _GAMMA / (_A * _A)


def _bf16(v):
    u = np.asarray(v, np.float32).view(np.uint32)
    r = (u + 0x7FFF + ((u >> 16) & 1)) & np.uint32(0xFFFF0000)
    return r.view(np.float32)


def _build_m():
    m = np.zeros((16, 256), dtype=np.float32)
    for j in (0, 1):
        for l in range(45):
            m[0 + j, 128 * j + l] = 5.0 * _KAPPA
            m[4 + j, 128 * j + l] = _KAPPA
            m[14, 128 * j + l] = -_KAPPA * l
        for l in range(45, 75):
            m[2 + j, 128 * j + l] = 5.0 * _KAPPA
            m[6 + j, 128 * j + l] = _KAPPA
            m[14, 128 * j + l] = -_KAPPA * (l - 45)
        for l in range(75, 107):
            m[8 + j, 128 * j + l] = _KAPPA
            m[14, 128 * j + l] = -_KAPPA * (l - 75)
        for l in range(107, 127):
            off = _A * 0.1 * (l - 107)
            hi = _bf16(off)
            m[10 + j, 128 * j + l] = _A
            m[12 + j, 128 * j + l] = _A
            m[14, 128 * j + l] = -hi
            m[15, 128 * j + l] = -_bf16(np.float32(off) - hi)
    return m


_M = _build_m()


def _body(x_ref, m_ref, w_ref, out_ref):
    q = jax.lax.dot_general(
        x_ref[...], m_ref[...], (((1,), (0,)), ((), ())),
        preferred_element_type=jnp.float32)
    g = jnp.exp(q * q * (-_S))
    out_ref[...] = jnp.dot(g, w_ref[...], preferred_element_type=jnp.float32)


@functools.partial(jax.jit, static_argnames=("interpret",))
def _run(x, m, w, *, interpret=False):
    grid = (_N // _BLOCK,)
    return pl.pallas_call(
        _body,
        grid=grid,
        in_specs=[pl.BlockSpec((_BLOCK, 16), lambda i: (i, 0)),
                  pl.BlockSpec((16, 256), lambda i: (0, 0)),
                  pl.BlockSpec((256, 128), lambda i: (0, 0))],
        out_specs=pl.BlockSpec((_BLOCK, 128), lambda i: (i, 0)),
        out_shape=jax.ShapeDtypeStruct((_N, 128), jnp.float32),
        interpret=interpret,
    )(x, m, w)


def kernel(atomic_num, hyb, heavydegree, heterodegree, smarts,
           emb_atomic_num, emb_hyb, emb_heavydegree, emb_heterodegree,
           emb_smarts, partialcharge, charge_W, charge_b, proj_W, proj_b,
           interpret=False):
    P = proj_W.T
    P0, P1 = P[:32], P[32:]
    t01 = (emb_atomic_num[:, None, :]
           + emb_heavydegree[None, :, :]).reshape(45, 32)
    t23 = (emb_hyb[:, None, :]
           + emb_heterodegree[None, :, :]).reshape(30, 32)
    tcat = jnp.concatenate([t01, t23, emb_smarts], axis=0)

    def fused_w(Pj, bias_row):
        return jnp.concatenate(
            [tcat @ Pj, charge_W.T @ Pj, bias_row.reshape(1, 128)], axis=0)

    w = jnp.concatenate(
        [fused_w(P0, charge_b @ P0 + proj_b), fused_w(P1, charge_b @ P1)],
        axis=0)

    f32 = jnp.float32
    pc_hi = partialcharge.astype(jnp.bfloat16).astype(f32)
    pc_lo = partialcharge - pc_hi
    ones = jnp.ones((_N, 2), f32)
    x = jnp.concatenate(
        [atomic_num.astype(f32), hyb.astype(f32), heavydegree.astype(f32),
         heterodegree.astype(f32), smarts.astype(f32), pc_hi, pc_lo, ones],
        axis=1)

    return _run(x, jnp.asarray(_M), w, interpret=interpret)
